# Initial kernel scaffold; baseline (speedup 1.0000x reference)
import jax
import jax.numpy as jnp
from jax.experimental import pallas as pl


def kernel(x, router_logits, wi_0, wi_1, wo):
    raise NotImplementedError("write your pallas kernel here")


# speedup vs baseline: 1.4255x; 1.4255x over previous
import functools

import jax
import jax.numpy as jnp
from jax.experimental import pallas as pl
from jax.experimental.pallas import tpu as pltpu

T = 128
H = 1024
FF = 2048
E = 16
BF = 512
NF = FF // BF


def _moe_body(rl_ref, x_ref, wi0_ref, wi1_ref, wo_ref, out_ref, w_ref):
    e = pl.program_id(0)
    f = pl.program_id(1)

    @pl.when((e == 0) & (f == 0))
    def _prologue():
        logits = rl_ref[...]
        lane = jax.lax.broadcasted_iota(jnp.int32, (T, E), 1)
        neg = jnp.float32(jnp.finfo(jnp.float32).min)
        m1 = jnp.max(logits, axis=-1, keepdims=True)
        i1 = jnp.min(jnp.where(logits == m1, lane, E), axis=-1, keepdims=True)
        masked = jnp.where(lane == i1, neg, logits)
        m2 = jnp.max(masked, axis=-1, keepdims=True)
        i2 = jnp.min(jnp.where(masked == m2, lane, E), axis=-1, keepdims=True)
        w1 = 1.0 / (1.0 + jnp.exp(m2 - m1))
        w2 = 1.0 - w1
        w_ref[...] = (jnp.where(lane == i1, w1, 0.0)
                      + jnp.where(lane == i2, w2, 0.0))
        out_ref[...] = jnp.zeros_like(out_ref)

    x = x_ref[...]
    h = (jax.nn.silu(jnp.dot(x, wi0_ref[0], preferred_element_type=jnp.float32))
         * jnp.dot(x, wi1_ref[0], preferred_element_type=jnp.float32))
    ye = jnp.dot(h, wo_ref[0], preferred_element_type=jnp.float32)

    lane = jax.lax.broadcasted_iota(jnp.int32, (T, E), 1)
    w_e = jnp.sum(jnp.where(lane == e, w_ref[...], 0.0), axis=-1, keepdims=True)
    out_ref[...] += ye * w_e


@functools.partial(jax.jit)
def kernel(x, router_logits, wi_0, wi_1, wo):
    return pl.pallas_call(
        _moe_body,
        grid=(E, NF),
        in_specs=[
            pl.BlockSpec((T, E), lambda e, f: (0, 0)),
            pl.BlockSpec((T, H), lambda e, f: (0, 0)),
            pl.BlockSpec((1, H, BF), lambda e, f: (e, 0, f)),
            pl.BlockSpec((1, H, BF), lambda e, f: (e, 0, f)),
            pl.BlockSpec((1, BF, H), lambda e, f: (e, f, 0)),
        ],
        out_specs=pl.BlockSpec((T, H), lambda e, f: (0, 0)),
        out_shape=jax.ShapeDtypeStruct((T, H), jnp.float32),
        scratch_shapes=[pltpu.VMEM((T, E), jnp.float32)],
    )(router_logits, x, wi_0, wi_1, wo)


# speedup vs baseline: 1.4524x; 1.0189x over previous
import functools

import jax
import jax.numpy as jnp
from jax.experimental import pallas as pl
from jax.experimental.pallas import tpu as pltpu

T = 128
H = 1024
FF = 2048
E = 16
BF = 2048
NF = FF // BF


def _moe_body(rl_ref, x_ref, wi0_ref, wi1_ref, wo_ref, out_ref, w_ref):
    e = pl.program_id(0)

    @pl.when(e == 0)
    def _prologue():
        logits = rl_ref[...]
        lane = jax.lax.broadcasted_iota(jnp.int32, (T, E), 1)
        neg = jnp.float32(jnp.finfo(jnp.float32).min)
        m1 = jnp.max(logits, axis=-1, keepdims=True)
        i1 = jnp.min(jnp.where(logits == m1, lane, E), axis=-1, keepdims=True)
        masked = jnp.where(lane == i1, neg, logits)
        m2 = jnp.max(masked, axis=-1, keepdims=True)
        i2 = jnp.min(jnp.where(masked == m2, lane, E), axis=-1, keepdims=True)
        w1 = 1.0 / (1.0 + jnp.exp(m2 - m1))
        w2 = 1.0 - w1
        w_ref[...] = (jnp.where(lane == i1, w1, 0.0)
                      + jnp.where(lane == i2, w2, 0.0))
        out_ref[...] = jnp.zeros_like(out_ref)

    x = x_ref[...]
    h = (jax.nn.silu(jnp.dot(x, wi0_ref[0], preferred_element_type=jnp.float32))
         * jnp.dot(x, wi1_ref[0], preferred_element_type=jnp.float32))
    ye = jnp.dot(h, wo_ref[0], preferred_element_type=jnp.float32)

    lane = jax.lax.broadcasted_iota(jnp.int32, (T, E), 1)
    w_e = jnp.sum(jnp.where(lane == e, w_ref[...], 0.0), axis=-1, keepdims=True)
    out_ref[...] += ye * w_e


@functools.partial(jax.jit)
def kernel(x, router_logits, wi_0, wi_1, wo):
    return pl.pallas_call(
        _moe_body,
        grid=(E,),
        in_specs=[
            pl.BlockSpec((T, E), lambda e: (0, 0)),
            pl.BlockSpec((T, H), lambda e: (0, 0)),
            pl.BlockSpec((1, H, BF), lambda e: (e, 0, 0)),
            pl.BlockSpec((1, H, BF), lambda e: (e, 0, 0)),
            pl.BlockSpec((1, BF, H), lambda e: (e, 0, 0)),
        ],
        out_specs=pl.BlockSpec((T, H), lambda e: (0, 0)),
        out_shape=jax.ShapeDtypeStruct((T, H), jnp.float32),
        scratch_shapes=[pltpu.VMEM((T, E), jnp.float32)],
    )(router_logits, x, wi_0, wi_1, wo)
